# baseline (speedup 1.0000x reference)
import functools

import jax
import jax.numpy as jnp
from jax import lax
from jax.experimental import pallas as pl
from jax.experimental.pallas import tpu as pltpu
from jax.experimental.pallas import tpu_sc as plsc

_BLK = 256
_NCHUNK = 4


def _select_kernel(W_ref, b_ref, ht_ref, hs_ref,
                   nht_ref, nhs_ref, idx_ref, ohT_ref):
    N = ht_ref.shape[0]
    K = idx_ref.shape[0]
    ht = ht_ref[:, :]
    s2 = jax.nn.sigmoid(
        jnp.dot(ht, W_ref[:, :], preferred_element_type=jnp.float32) + b_ref[0, 0]
    )
    sr = jnp.transpose(s2)
    ones_col = jnp.ones((N, 1), jnp.bfloat16)
    blocks = []
    for bi in range(N // _BLK):
        col = s2[bi * _BLK:(bi + 1) * _BLK, :]
        srb = jnp.broadcast_to(sr, (_BLK, N))
        colb = jnp.broadcast_to(col, (_BLK, N))
        j_ids = jax.lax.broadcasted_iota(jnp.int32, (_BLK, N), 1)
        i_ids = jax.lax.broadcasted_iota(jnp.int32, (_BLK, N), 0) + bi * _BLK
        beats = (srb > colb) | ((srb == colb) & (j_ids < i_ids))
        blocks.append(jnp.dot(beats.astype(jnp.bfloat16), ones_col,
                              preferred_element_type=jnp.float32))
    rank = jnp.concatenate(blocks, axis=0).astype(jnp.int32)
    rank_row = jnp.transpose(rank)
    kn_iota = jax.lax.broadcasted_iota(jnp.int32, (K, N), 0)
    oh = (jnp.broadcast_to(rank_row, (K, N)) == kn_iota).astype(jnp.float32)
    nk_iota = jax.lax.broadcasted_iota(jnp.int32, (N, K), 1)
    ohT_ref[:, :] = (jnp.broadcast_to(rank, (N, K)) == nk_iota).astype(jnp.bfloat16)
    nht_ref[:, :] = jnp.dot(oh, ht * s2, preferred_element_type=jnp.float32)
    nhs_ref[:, :] = jnp.dot(oh, hs_ref[:, :] * s2, preferred_element_type=jnp.float32)
    iota_col = jax.lax.broadcasted_iota(jnp.int32, (N, 1), 0).astype(jnp.float32)
    idx_ref[:, :] = jnp.dot(oh, iota_col,
                            preferred_element_type=jnp.float32).astype(jnp.int32)


def _adj_kernel(ohT_ref, bm_ref, g_ref, out_ref, gbuf, sems):
    N = ohT_ref.shape[0]
    K = out_ref.shape[0]
    rows = N // _NCHUNK
    copies = []
    for i in range(_NCHUNK):
        c = pltpu.make_async_copy(
            g_ref.at[pl.ds(i * rows, rows), :],
            gbuf.at[pl.ds(i * rows, rows), :],
            sems.at[i])
        c.start()
        copies.append(c)
    bm = bm_ref[:, :].astype(jnp.bfloat16)
    ohT_b = ohT_ref[:, :]
    m = None
    for i in range(_NCHUNK):
        copies[i].wait()
        gc = gbuf[pl.ds(i * rows, rows), :].astype(jnp.bfloat16)
        cm_i = jnp.dot(gc, ohT_b,
                       preferred_element_type=jnp.float32).astype(jnp.bfloat16)
        part = jnp.dot(bm[:, i * rows:(i + 1) * rows], cm_i,
                       preferred_element_type=jnp.float32)
        m = part if m is None else m + part
    un_g = (m != 0).astype(jnp.float32)
    ones = jnp.ones((1, K), jnp.float32)
    deg_row = jax.lax.dot_general(
        ones, un_g, (((1,), (1,)), ((), ())),
        preferred_element_type=jnp.float32)
    out_ref[:, :] = un_g / deg_row


def _sc_gather(g, idx1d, K, N):
    info = plsc.get_sparse_core_info()
    NC, NS = info.num_cores, info.num_subcores
    NW = NC * NS
    b_per_w = K // NW
    mesh = plsc.VectorSubcoreMesh(core_axis_name="c", subcore_axis_name="s")

    @functools.partial(
        pl.kernel, mesh=mesh,
        out_type=jax.ShapeDtypeStruct((K, N), jnp.int32),
        scratch_types=[
            pltpu.VMEM((b_per_w,), jnp.int32),
            pltpu.VMEM((b_per_w, N), jnp.int32),
            pltpu.SemaphoreType.DMA,
        ],
    )
    def gather_rows(table_hbm, idx_hbm, out_hbm, idx_v, rows_v, sem):
        wid = lax.axis_index("s") * NC + lax.axis_index("c")
        base = wid * b_per_w
        pltpu.sync_copy(idx_hbm.at[pl.ds(base, b_per_w)], idx_v)
        pltpu.async_copy(table_hbm.at[idx_v], rows_v, sem).wait()
        pltpu.sync_copy(rows_v, out_hbm.at[pl.ds(base, b_per_w)])

    return gather_rows(g, idx1d)


def kernel(ht, hs, g, k, W, b):
    N, D = ht.shape
    K = max(2, 1024)
    b2 = jnp.asarray(b, jnp.float32).reshape(1, 1)
    nht, nhs, idx, ohT_b = pl.pallas_call(
        _select_kernel,
        out_shape=[
            jax.ShapeDtypeStruct((K, D), jnp.float32),
            jax.ShapeDtypeStruct((K, D), jnp.float32),
            jax.ShapeDtypeStruct((K, 1), jnp.int32),
            jax.ShapeDtypeStruct((N, K), jnp.bfloat16),
        ],
    )(W, b2, ht, hs)
    bm = _sc_gather(g, idx.reshape(K), K, N)
    g_norm = pl.pallas_call(
        _adj_kernel,
        in_specs=[
            pl.BlockSpec(memory_space=pltpu.MemorySpace.VMEM),
            pl.BlockSpec(memory_space=pltpu.MemorySpace.VMEM),
            pl.BlockSpec(memory_space=pltpu.MemorySpace.HBM),
        ],
        out_shape=jax.ShapeDtypeStruct((K, K), jnp.float32),
        scratch_shapes=[
            pltpu.VMEM((N, N), jnp.int32),
            pltpu.SemaphoreType.DMA((_NCHUNK,)),
        ],
    )(ohT_b, bm, g)
    return nht, nhs, g_norm


# speedup vs baseline: 1.7085x; 1.7085x over previous
import jax
import jax.numpy as jnp
from jax.experimental import pallas as pl
from jax.experimental.pallas import tpu as pltpu

_BLK = 256
_NCHUNK = 4


def _gnn_kernel(W_ref, b_ref, ht_ref, hs_ref, g_ref,
                nht_ref, nhs_ref, out_ref, gbuf, sems):
    N = ht_ref.shape[0]
    K = out_ref.shape[0]
    rows = N // _NCHUNK

    def _start(i):
        c = pltpu.make_async_copy(
            g_ref.at[pl.ds(i * rows, rows), :],
            gbuf.at[i % 2],
            sems.at[i % 2])
        c.start()
        return c

    copies = [_start(0), _start(1)]
    ht = ht_ref[:, :]
    s2 = jax.nn.sigmoid(
        jnp.dot(ht, W_ref[:, :], preferred_element_type=jnp.float32) + b_ref[0, 0]
    )
    sr = jnp.transpose(s2)
    ones_col = jnp.ones((N, 1), jnp.bfloat16)
    blocks = []
    for bi in range(N // _BLK):
        col = s2[bi * _BLK:(bi + 1) * _BLK, :]
        srb = jnp.broadcast_to(sr, (_BLK, N))
        colb = jnp.broadcast_to(col, (_BLK, N))
        j_ids = jax.lax.broadcasted_iota(jnp.int32, (_BLK, N), 1)
        i_ids = jax.lax.broadcasted_iota(jnp.int32, (_BLK, N), 0) + bi * _BLK
        beats = (srb > colb) | ((srb == colb) & (j_ids < i_ids))
        blocks.append(jnp.dot(beats.astype(jnp.bfloat16), ones_col,
                              preferred_element_type=jnp.float32))
    rank = jnp.concatenate(blocks, axis=0).astype(jnp.int32)
    rank_row = jnp.transpose(rank)
    kn_iota = jax.lax.broadcasted_iota(jnp.int32, (K, N), 0)
    oh = (jnp.broadcast_to(rank_row, (K, N)) == kn_iota).astype(jnp.float32)
    nk_iota = jax.lax.broadcasted_iota(jnp.int32, (N, K), 1)
    ohT_b = (jnp.broadcast_to(rank, (N, K)) == nk_iota).astype(jnp.int8)
    nht_ref[:, :] = jnp.dot(oh, ht * s2, preferred_element_type=jnp.float32)
    nhs_ref[:, :] = jnp.dot(oh, hs_ref[:, :] * s2, preferred_element_type=jnp.float32)
    oh_b = oh.astype(jnp.int8)
    cm_chunks = []
    bm_acc = None
    for i in range(_NCHUNK):
        copies[i].wait()
        gc = gbuf[i % 2].astype(jnp.int8)
        if i + 2 < _NCHUNK:
            copies.append(_start(i + 2))
        cm_chunks.append(
            jnp.dot(gc, ohT_b,
                    preferred_element_type=jnp.int32).astype(jnp.int8))
        part = jnp.dot(oh_b[:, i * rows:(i + 1) * rows], gc,
                       preferred_element_type=jnp.int32)
        bm_acc = part if bm_acc is None else bm_acc + part
    bm = bm_acc.astype(jnp.int8)
    m = None
    for i in range(_NCHUNK):
        part = jnp.dot(bm[:, i * rows:(i + 1) * rows], cm_chunks[i],
                       preferred_element_type=jnp.int32)
        m = part if m is None else m + part
    un_g = (m != 0).astype(jnp.float32)
    ones = jnp.ones((1, K), jnp.float32)
    deg_row = jax.lax.dot_general(
        ones, un_g, (((1,), (1,)), ((), ())),
        preferred_element_type=jnp.float32)
    out_ref[:, :] = un_g / deg_row


def kernel(ht, hs, g, k, W, b):
    N, D = ht.shape
    K = max(2, 1024)
    b2 = jnp.asarray(b, jnp.float32).reshape(1, 1)
    nht, nhs, g_norm = pl.pallas_call(
        _gnn_kernel,
        in_specs=[
            pl.BlockSpec(memory_space=pltpu.MemorySpace.VMEM),
            pl.BlockSpec(memory_space=pltpu.MemorySpace.VMEM),
            pl.BlockSpec(memory_space=pltpu.MemorySpace.VMEM),
            pl.BlockSpec(memory_space=pltpu.MemorySpace.VMEM),
            pl.BlockSpec(memory_space=pltpu.MemorySpace.HBM),
        ],
        out_shape=[
            jax.ShapeDtypeStruct((K, D), jnp.float32),
            jax.ShapeDtypeStruct((K, D), jnp.float32),
            jax.ShapeDtypeStruct((K, K), jnp.float32),
        ],
        scratch_shapes=[
            pltpu.VMEM((2, N // _NCHUNK, N), jnp.int32),
            pltpu.SemaphoreType.DMA((2,)),
        ],
    )(W, b2, ht, hs, g)
    return nht, nhs, g_norm


# speedup vs baseline: 1.7822x; 1.0431x over previous
import jax
import jax.numpy as jnp
from jax.experimental import pallas as pl
from jax.experimental.pallas import tpu as pltpu

_BLK = 256
_NCHUNK = 4


def _gnn_kernel(ht_ref, hs_ref, g_ref, W_ref, b_ref,
                nht_ref, nhs_ref, out_ref, gbuf, sems):
    N = ht_ref.shape[0]
    K = out_ref.shape[0]
    rows = N // _NCHUNK
    copies = []
    for i in range(_NCHUNK):
        c = pltpu.make_async_copy(
            g_ref.at[pl.ds(i * rows, rows), :],
            gbuf.at[pl.ds(i * rows, rows), :],
            sems.at[i])
        c.start()
        copies.append(c)
    ht = ht_ref[:, :]
    hs = hs_ref[:, :]
    s2 = jax.nn.sigmoid(
        jnp.dot(ht, W_ref[:, :], preferred_element_type=jnp.float32) + b_ref[0, 0]
    )
    sr = jnp.transpose(s2)
    blocks = []
    for bi in range(N // _BLK):
        col = s2[bi * _BLK:(bi + 1) * _BLK, :]
        srb = jnp.broadcast_to(sr, (_BLK, N))
        colb = jnp.broadcast_to(col, (_BLK, N))
        j_ids = jax.lax.broadcasted_iota(jnp.int32, (_BLK, N), 1)
        i_ids = jax.lax.broadcasted_iota(jnp.int32, (_BLK, N), 0) + bi * _BLK
        beats = (srb > colb) | ((srb == colb) & (j_ids < i_ids))
        blocks.append(jnp.sum(beats.astype(jnp.float32), axis=1, keepdims=True))
    rank = jnp.concatenate(blocks, axis=0).astype(jnp.int32)
    rank_row = jnp.transpose(rank)
    kn_iota = jax.lax.broadcasted_iota(jnp.int32, (K, N), 0)
    oh = (jnp.broadcast_to(rank_row, (K, N)) == kn_iota).astype(jnp.float32)
    nk_iota = jax.lax.broadcasted_iota(jnp.int32, (N, K), 1)
    ohT_b = (jnp.broadcast_to(rank, (N, K)) == nk_iota).astype(jnp.bfloat16)
    nht_ref[:, :] = jnp.dot(oh, ht * s2, preferred_element_type=jnp.float32)
    nhs_ref[:, :] = jnp.dot(oh, hs * s2, preferred_element_type=jnp.float32)
    oh_b = oh.astype(jnp.bfloat16)
    gb_chunks, cm_chunks = [], []
    for i in range(_NCHUNK):
        copies[i].wait()
        gc = gbuf[pl.ds(i * rows, rows), :].astype(jnp.bfloat16)
        gb_chunks.append(gc)
        cm_chunks.append(
            jnp.dot(gc, ohT_b,
                    preferred_element_type=jnp.float32).astype(jnp.bfloat16))
    gb = jnp.concatenate(gb_chunks, axis=0)
    cm = jnp.concatenate(cm_chunks, axis=0)
    bm = jnp.dot(oh_b, gb,
                 preferred_element_type=jnp.float32).astype(jnp.bfloat16)
    m = jnp.dot(bm, cm, preferred_element_type=jnp.float32)
    un_g = (m != 0).astype(jnp.float32)
    ones = jnp.ones((1, K), jnp.float32)
    deg_row = jax.lax.dot_general(
        ones, un_g, (((1,), (1,)), ((), ())),
        preferred_element_type=jnp.float32)
    out_ref[:, :] = un_g / deg_row


def kernel(ht, hs, g, k, W, b):
    N, D = ht.shape
    K = max(2, 1024)
    b2 = jnp.asarray(b, jnp.float32).reshape(1, 1)
    nht, nhs, g_norm = pl.pallas_call(
        _gnn_kernel,
        in_specs=[
            pl.BlockSpec(memory_space=pltpu.MemorySpace.VMEM),
            pl.BlockSpec(memory_space=pltpu.MemorySpace.VMEM),
            pl.BlockSpec(memory_space=pltpu.MemorySpace.HBM),
            pl.BlockSpec(memory_space=pltpu.MemorySpace.VMEM),
            pl.BlockSpec(memory_space=pltpu.MemorySpace.VMEM),
        ],
        out_shape=[
            jax.ShapeDtypeStruct((K, D), jnp.float32),
            jax.ShapeDtypeStruct((K, D), jnp.float32),
            jax.ShapeDtypeStruct((K, K), jnp.float32),
        ],
        scratch_shapes=[
            pltpu.VMEM((N, N), jnp.int32),
            pltpu.SemaphoreType.DMA((_NCHUNK,)),
        ],
    )(ht, hs, g, W, b2)
    return nht, nhs, g_norm
